# baseline (speedup 1.0000x reference)
import functools

import jax
import jax.numpy as jnp
from jax import lax
from jax.experimental import pallas as pl
from jax.experimental.pallas import tpu as pltpu
from jax.experimental.pallas import tpu_sc as plsc

_L = 16
_NC = 2
_NS = 16
_NW = _NC * _NS
_FEAT = 128
_CHUNK = 32768
_NACC = _FEAT // _L


def _sc_partials(x_flat, weights):
    total = x_flat.shape[0]
    per_w = total // _NW
    nchunk = per_w // _CHUNK
    mesh = plsc.VectorSubcoreMesh(core_axis_name="c", subcore_axis_name="s")

    @functools.partial(
        pl.kernel,
        mesh=mesh,
        out_type=jax.ShapeDtypeStruct((_NW, _L), jnp.float32),
        scratch_types=[
            pltpu.VMEM((_CHUNK,), jnp.float32),
            pltpu.VMEM((_CHUNK,), jnp.float32),
            pltpu.VMEM((_FEAT,), jnp.float32),
            pltpu.VMEM((_L,), jnp.float32),
            pltpu.SemaphoreType.DMA,
            pltpu.SemaphoreType.DMA,
        ],
    )
    def body(x_hbm, w_hbm, out_hbm, buf0, buf1, w_v, res_v, sem0, sem1):
        wid = lax.axis_index("s") * _NC + lax.axis_index("c")
        base = wid * per_w
        bufs = (buf0, buf1)
        sems = (sem0, sem1)

        pltpu.sync_copy(w_hbm, w_v)

        copies = [None, None]
        copies[0] = pltpu.async_copy(
            x_hbm.at[pl.ds(base, _CHUNK)], buf0, sem0)

        accs = tuple(jnp.zeros((_L,), jnp.float32) for _ in range(_NACC))
        for g in range(nchunk):
            if g + 1 < nchunk:
                copies[(g + 1) % 2] = pltpu.async_copy(
                    x_hbm.at[pl.ds(base + (g + 1) * _CHUNK, _CHUNK)],
                    bufs[(g + 1) % 2], sems[(g + 1) % 2])
            copies[g % 2].wait()
            buf = bufs[g % 2]

            def inner(j, a):
                row = j * _FEAT
                return tuple(
                    a[k] + buf[pl.ds(row + k * _L, _L)] for k in range(_NACC))

            accs = lax.fori_loop(0, _CHUNK // _FEAT, inner, accs)

        res = jnp.zeros((_L,), jnp.float32)
        for k in range(_NACC):
            res = res + accs[k] * w_v[pl.ds(k * _L, _L)]
        res_v[...] = res
        pltpu.sync_copy(res_v, out_hbm.at[wid])

    return body(x_flat, weights)


def _finish_body(p_ref, o_ref):
    o_ref[...] = jnp.sum(p_ref[...]).reshape(1, 1)


def kernel(input_observation, weights):
    batch, feat = input_observation.shape
    partials = _sc_partials(input_observation.reshape(-1), weights)
    out = pl.pallas_call(
        _finish_body,
        out_shape=jax.ShapeDtypeStruct((1, 1), jnp.float32),
    )(partials.reshape(8, 64))
    return out[0, 0]
